# Initial kernel scaffold; baseline (speedup 1.0000x reference)
import jax
import jax.numpy as jnp
from jax.experimental import pallas as pl


def kernel(x, edge_index, batch, W1, b1, W2, b2, W3, b3, lin1_W, lin1_b, lin2_W, lin2_b):
    raise NotImplementedError("write your pallas kernel here")


# speedup vs baseline: 5.6371x; 5.6371x over previous
import functools

import jax
import jax.numpy as jnp
from jax import lax
from jax.experimental import pallas as pl
from jax.experimental.pallas import tpu as pltpu
from jax.experimental.pallas import tpu_sc as plsc

NN = 10000
EE = 320000
HID = 256
HH = 128
NG = 256

NSUB = 16
EPT = EE // NSUB
EK = 80
NIT = EPT // EK

ROWB = 400
NTILE = 624
NREM = NN - NSUB * NTILE

_mesh = plsc.VectorSubcoreMesh(core_axis_name="c", subcore_axis_name="s")


def _sc_mp_body(src2_hbm, dst_hbm, y2_hbm, zero_hbm,
                out_hbm, sidx, didx, rows, acc, sem):
    c = lax.axis_index("c")
    s = lax.axis_index("s")

    pltpu.sync_copy(zero_hbm.at[pl.ds(s * NTILE, NTILE)],
                    acc.at[pl.ds(s * NTILE, NTILE)])

    @pl.when(s == 0)
    def _():
        pltpu.sync_copy(zero_hbm.at[pl.ds(NSUB * NTILE, NREM)],
                        acc.at[pl.ds(NSUB * NTILE, NREM)])

    plsc.subcore_barrier()

    def step(i, carry):
        base = pl.multiple_of(s * EPT + i * EK, 8)
        sbase = pl.multiple_of(c * EE + base, 8)
        pltpu.sync_copy(src2_hbm.at[pl.ds(sbase, EK)], sidx)
        pltpu.sync_copy(dst_hbm.at[pl.ds(base, EK)], didx)
        pltpu.async_copy(y2_hbm.at[sidx], rows, sem).wait()
        pltpu.sync_copy(rows, acc.at[didx], add=True)
        return carry

    lax.fori_loop(0, NIT, step, 0)
    plsc.subcore_barrier()

    obase = pl.multiple_of(c * NN + s * NTILE, 8)
    pltpu.sync_copy(acc.at[pl.ds(s * NTILE, NTILE)],
                    out_hbm.at[pl.ds(obase, NTILE)])

    @pl.when(s == 0)
    def _():
        tbase = pl.multiple_of(c * NN + NSUB * NTILE, 8)
        pltpu.sync_copy(acc.at[pl.ds(NSUB * NTILE, NREM)],
                        out_hbm.at[pl.ds(tbase, NREM)])


_sc_mp = pl.kernel(
    _sc_mp_body, mesh=_mesh,
    out_type=jax.ShapeDtypeStruct((2 * NN, HH), jnp.float32),
    scratch_types=[pltpu.VMEM((EK,), jnp.int32),
                   pltpu.VMEM((EK,), jnp.int32),
                   pltpu.VMEM((EK, HH), jnp.float32),
                   pltpu.VMEM_SHARED((NN, HH), jnp.float32),
                   pltpu.SemaphoreType.DMA],
)


def _mm_scale_body(h_ref, w_ref, deg_ref, y2_ref):
    dinv = lax.rsqrt(deg_ref[:, 0:1] + 1.0)
    y = jnp.dot(h_ref[...], w_ref[...], preferred_element_type=jnp.float32)
    y2_ref[...] = y * dinv


def _mm_scale(h, w, deg16):
    din = h.shape[1]
    nb = NN // ROWB
    return pl.pallas_call(
        _mm_scale_body,
        grid=(2, nb),
        in_specs=[pl.BlockSpec((ROWB, din), lambda k, i: (i, 0)),
                  pl.BlockSpec((din, HH), lambda k, i: (0, k)),
                  pl.BlockSpec((ROWB, 16), lambda k, i: (i, 0))],
        out_specs=pl.BlockSpec((ROWB, HH), lambda k, i: (k * nb + i, 0)),
        out_shape=jax.ShapeDtypeStruct((2 * NN, HH), jnp.float32),
    )(h, w, deg16)


def _combine_body(alo_ref, ahi_ref, ylo_ref, yhi_ref, deg_ref, b_ref, o_ref):
    dinv = lax.rsqrt(deg_ref[:, 0:1] + 1.0)
    lo = (alo_ref[...] + ylo_ref[...]) * dinv + b_ref[:, :HH]
    hi = (ahi_ref[...] + yhi_ref[...]) * dinv + b_ref[:, HH:]
    o_ref[...] = jnp.maximum(jnp.concatenate([lo, hi], axis=1), 0.0)


def _combine(acc2, y2, deg16, b2d):
    nb = NN // ROWB
    return pl.pallas_call(
        _combine_body,
        grid=(nb,),
        in_specs=[pl.BlockSpec((ROWB, HH), lambda i: (i, 0)),
                  pl.BlockSpec((ROWB, HH), lambda i: (nb + i, 0)),
                  pl.BlockSpec((ROWB, HH), lambda i: (i, 0)),
                  pl.BlockSpec((ROWB, HH), lambda i: (nb + i, 0)),
                  pl.BlockSpec((ROWB, 16), lambda i: (i, 0)),
                  pl.BlockSpec((1, HID), lambda i: (0, 0))],
        out_specs=pl.BlockSpec((ROWB, HID), lambda i: (i, 0)),
        out_shape=jax.ShapeDtypeStruct((NN, HID), jnp.float32),
    )(acc2, acc2, y2, y2, deg16, b2d)


def _pool_body(h_ref, b_ref, sums_ref, cnt_ref):
    @pl.when(pl.program_id(0) == 0)
    def _():
        sums_ref[...] = jnp.zeros_like(sums_ref)
        cnt_ref[...] = jnp.zeros_like(cnt_ref)

    seg = b_ref[:, 0]
    gid = lax.broadcasted_iota(jnp.int32, (NG, ROWB), 0)
    onehot = (gid == seg[None, :]).astype(jnp.float32)
    sums_ref[...] += jnp.dot(onehot, h_ref[...],
                             preferred_element_type=jnp.float32)
    cnt = jnp.sum(onehot, axis=1, keepdims=True)
    cnt_ref[...] += jnp.broadcast_to(cnt, (NG, 16))


def _pool(h, batch2d):
    return pl.pallas_call(
        _pool_body,
        grid=(NN // ROWB,),
        in_specs=[pl.BlockSpec((ROWB, HID), lambda i: (i, 0)),
                  pl.BlockSpec((ROWB, 1), lambda i: (i, 0))],
        out_specs=[pl.BlockSpec((NG, HID), lambda i: (0, 0)),
                   pl.BlockSpec((NG, 16), lambda i: (0, 0))],
        out_shape=[jax.ShapeDtypeStruct((NG, HID), jnp.float32),
                   jax.ShapeDtypeStruct((NG, 16), jnp.float32)],
    )(h, batch2d)


def _head_body(s_ref, c_ref, w1_ref, b1_ref, w2_ref, b2_ref, o_ref):
    pooled = s_ref[...] / jnp.maximum(c_ref[:, 0:1], 1.0)
    hh = jnp.maximum(
        jnp.dot(pooled, w1_ref[...], preferred_element_type=jnp.float32)
        + b1_ref[...], 0.0)
    o_ref[...] = (jnp.dot(hh, w2_ref[...], preferred_element_type=jnp.float32)
                  + b2_ref[...])


def _head(sums, cnt16, w1, b1, w2, b2):
    return pl.pallas_call(
        _head_body,
        out_shape=jax.ShapeDtypeStruct((NG, 1), jnp.float32),
    )(sums, cnt16, w1, b1, w2, b2)


def kernel(x, edge_index, batch, W1, b1, W2, b2, W3, b3,
           lin1_W, lin1_b, lin2_W, lin2_b):
    src = edge_index[0]
    dst = edge_index[1]
    src2 = jnp.concatenate([src, src + NN])
    zeros128 = jnp.zeros((NN, HH), jnp.float32)
    ones2n = jnp.ones((2 * NN, HH), jnp.float32)

    deg16 = _sc_mp(src2, dst, ones2n, zeros128)[:NN, :16]

    h = x
    for w, b in ((W1, b1), (W2, b2), (W3, b3)):
        y2 = _mm_scale(h, w, deg16)
        acc2 = _sc_mp(src2, dst, y2, zeros128)
        h = _combine(acc2, y2, deg16, b.reshape(1, HID))

    sums, cnt16 = _pool(h, batch.reshape(NN, 1))
    out = _head(sums, cnt16, lin1_W, lin1_b.reshape(1, HID),
                lin2_W, lin2_b.reshape(1, 1))
    return out.reshape(-1)
